# baseline (speedup 1.0000x reference)
import jax
import jax.numpy as jnp
from jax import lax
from jax.experimental import pallas as pl
from jax.experimental.pallas import tpu as pltpu
from jax.experimental.pallas import tpu_sc as plsc

VOCAB = 16384
GRP = 128
LANE = 16


def _sc_diag_tiles(rows, seq):
    n_groups = rows // GRP
    info = plsc.get_sparse_core_info()
    ns = info.num_subcores
    mesh = plsc.VectorSubcoreMesh(
        core_axis_name="c", subcore_axis_name="s", num_cores=1
    )
    nw = ns
    g_per_w = -(-n_groups // nw)

    def body(d_hbm, stage, sem):
        wid = lax.axis_index("s")
        lanes = lax.iota(jnp.int32, LANE)
        zeros16 = jnp.zeros((LANE,), jnp.float32)
        for k in range(g_per_w):
            g = wid + k * nw

            @pl.when(g < n_groups)
            def _(k=k, g=g):
                seq0 = lax.rem(g * GRP, seq)

                @pl.loop(0, GRP)
                def _(j):
                    val = 1.0 + 0.1 * (seq0 + j).astype(jnp.float32)
                    vline = jnp.where(lanes == lax.rem(j, LANE), val, 0.0)
                    jc = lax.div(j, LANE)
                    for c in range(GRP // LANE):
                        stage[k, j, pl.ds(c * LANE, LANE)] = jnp.where(
                            jc == c, vline, zeros16
                        )

                pltpu.async_copy(stage.at[k], d_hbm.at[g], sem)

        for k in range(g_per_w):
            g = wid + k * nw

            @pl.when(g < n_groups)
            def _(k=k):
                pltpu.make_async_copy(stage.at[k], d_hbm.at[0], sem).wait()

    fn = pl.kernel(
        body,
        out_type=jax.ShapeDtypeStruct((n_groups, GRP, GRP), jnp.float32),
        mesh=mesh,
        scratch_types=[
            pltpu.VMEM((g_per_w, GRP, GRP), jnp.float32),
            pltpu.SemaphoreType.DMA,
        ],
    )
    return fn()


def _fill_patch_kernel(seq, d_ref, out_ref):
    g = pl.program_id(0)
    seq0 = lax.rem(g * GRP, seq)
    out_ref[...] = jnp.zeros_like(out_ref)
    out_ref[:, pl.ds(seq0, GRP)] = d_ref[0]


def _tc_fill(tiles, rows, seq):
    from functools import partial

    return pl.pallas_call(
        partial(_fill_patch_kernel, seq),
        grid=(rows // GRP,),
        in_specs=[pl.BlockSpec((1, GRP, GRP), lambda g: (g, 0, 0))],
        out_specs=pl.BlockSpec((GRP, VOCAB), lambda g: (g, 0)),
        out_shape=jax.ShapeDtypeStruct((rows, VOCAB), jnp.float32),
    )(tiles)


def kernel(input_ids):
    batch, seq = input_ids.shape
    rows = batch * seq
    tiles = _sc_diag_tiles(rows, seq)
    out = _tc_fill(tiles, rows, seq)
    return out.reshape(batch, seq, VOCAB)


# speedup vs baseline: 1.0474x; 1.0474x over previous
import jax
import jax.numpy as jnp
from jax import lax
from jax.experimental import pallas as pl
from jax.experimental.pallas import tpu as pltpu
from jax.experimental.pallas import tpu_sc as plsc

VOCAB = 16384
ROW_BLK = 128
GRP = 128
LANE = 16


def _zero_kernel(out_ref):
    out_ref[...] = jnp.zeros_like(out_ref)


def _tc_zeros(rows):
    return pl.pallas_call(
        _zero_kernel,
        grid=(rows // ROW_BLK,),
        out_specs=pl.BlockSpec((ROW_BLK, VOCAB), lambda i: (i, 0)),
        out_shape=jax.ShapeDtypeStruct((rows, VOCAB), jnp.float32),
    )()


def _build_tile(stage, k, seq0):
    lanes = lax.iota(jnp.int32, LANE)
    zeros16 = jnp.zeros((LANE,), jnp.float32)

    @pl.loop(0, GRP)
    def _(j):
        val = 1.0 + 0.1 * (seq0 + j).astype(jnp.float32)
        vline = jnp.where(lanes == lax.rem(j, LANE), val, 0.0)
        jc = lax.div(j, LANE)
        for c in range(GRP // LANE):
            stage[k, j, pl.ds(c * LANE, LANE)] = jnp.where(jc == c, vline, zeros16)


def _sc_scatter(out_ref, rows, seq):
    n_groups = rows // GRP
    info = plsc.get_sparse_core_info()
    ns = info.num_subcores
    mesh = plsc.VectorSubcoreMesh(
        core_axis_name="c", subcore_axis_name="s", num_cores=1
    )
    nw = ns
    g_per_w = -(-n_groups // nw)
    shared_tile = (nw * GRP) % seq == 0
    n_stage = 1 if shared_tile else g_per_w

    def body(out_hbm, stage, sem):
        wid = lax.axis_index("s")
        if shared_tile:
            _build_tile(stage, 0, wid * GRP % seq)
        for k in range(g_per_w):
            g = wid + k * nw

            @pl.when(g < n_groups)
            def _(k=k, g=g):
                row0 = g * GRP
                if not shared_tile:
                    _build_tile(stage, k, lax.rem(row0, seq))
                pltpu.async_copy(
                    stage.at[0 if shared_tile else k],
                    out_hbm.at[pl.ds(row0, GRP), pl.ds(lax.rem(row0, seq), GRP)],
                    sem,
                )

        for k in range(g_per_w):
            g = wid + k * nw

            @pl.when(g < n_groups)
            def _():
                pltpu.make_async_copy(
                    stage.at[0],
                    out_hbm.at[pl.ds(0, GRP), pl.ds(0, GRP)],
                    sem,
                ).wait()

    fn = pl.kernel(
        body,
        out_type=(),
        mesh=mesh,
        scratch_types=[
            pltpu.VMEM((n_stage, GRP, GRP), jnp.float32),
            pltpu.SemaphoreType.DMA,
        ],
    )
    fn(out_ref)


def kernel(input_ids):
    batch, seq = input_ids.shape
    rows = batch * seq
    zeros = _tc_zeros(rows)
    ref = jax.new_ref(zeros)
    _sc_scatter(ref, rows, seq)
    return jax.freeze(ref).reshape(batch, seq, VOCAB)
